# Initial kernel scaffold; baseline (speedup 1.0000x reference)
import jax
import jax.numpy as jnp
from jax.experimental import pallas as pl


def kernel(x, W):
    raise NotImplementedError("write your pallas kernel here")


# speedup vs baseline: 4.2625x; 4.2625x over previous
import functools

import jax
import jax.numpy as jnp
from jax import lax
from jax.experimental import pallas as pl
from jax.experimental.pallas import tpu as pltpu
from jax.experimental.pallas import tpu_sc as plsc

NC = 2
NS = 16
L = 16
NW = NC * NS

D = 10
CHUNK = 5120
PHASES = 10


@functools.lru_cache(maxsize=None)
def _build(n_idx: int):
    per_w = n_idx // NW
    iters = per_w // CHUNK
    groups = CHUNK // L

    mesh = plsc.VectorSubcoreMesh(core_axis_name="c", subcore_axis_name="s")

    @functools.partial(
        pl.kernel,
        mesh=mesh,
        out_type=jax.ShapeDtypeStruct((n_idx * D,), jnp.float32),
        scratch_types=[
            pltpu.VMEM((CHUNK,), jnp.int32),
            pltpu.VMEM((CHUNK * D,), jnp.float32),
            pltpu.VMEM((100,), jnp.float32),
        ],
        compiler_params=pltpu.CompilerParams(needs_layout_passes=False),
    )
    def k(idx_hbm, tbl_hbm, out_hbm, idx_v, out_v, tbl_v):
        wid = lax.axis_index("s") * NC + lax.axis_index("c")
        base = wid * per_w

        pltpu.sync_copy(tbl_hbm, tbl_v)

        lane = lax.iota(jnp.int32, L)
        rows = []
        cols = []
        for v in range(PHASES):
            j = lane + (L * v)
            r = lax.shift_right_logical(j * 6554, 16)
            rows.append(r)
            cols.append(j - r * D)

        gather_dnums = lax.GatherDimensionNumbers(
            offset_dims=(), collapsed_slice_dims=(0,), start_index_map=(0,)
        )

        def group_body(g, carry):
            vidx = idx_v[pl.ds(g * L, L)]
            s = vidx * D
            gbase = g * (L * D)
            for v in range(PHASES):
                r = lax.gather(
                    s,
                    rows[v][:, None],
                    gather_dnums,
                    slice_sizes=(1,),
                    mode=lax.GatherScatterMode.PROMISE_IN_BOUNDS,
                )
                addr = r + cols[v]
                val = plsc.load_gather(tbl_v, [addr])
                out_v[pl.ds(gbase + v * L, L)] = val + val
            return carry

        for it in range(iters):
            off = base + it * CHUNK
            pltpu.sync_copy(idx_hbm.at[pl.ds(off, CHUNK)], idx_v)
            lax.fori_loop(0, groups, group_body, 0)
            pltpu.sync_copy(out_v, out_hbm.at[pl.ds(off * D, CHUNK * D)])

    return k


def kernel(x, W):
    b, t = x.shape
    n_idx = b * t
    xf = x.reshape(-1).astype(jnp.int32)
    wf = W.reshape(-1).astype(jnp.float32)
    out = _build(n_idx)(xf, wf)
    return out.reshape(b, t, D)


# speedup vs baseline: 28.4247x; 6.6685x over previous
import functools

import jax
import jax.numpy as jnp
from jax import lax
from jax.experimental import pallas as pl
from jax.experimental.pallas import tpu as pltpu
from jax.experimental.pallas import tpu_sc as plsc

NC = 2
NS = 16
L = 16
NW = NC * NS

D = 10
TILE = 1024
GRP = 4
BLK = GRP * TILE


@functools.lru_cache(maxsize=None)
def _build(t8n: int, bbn: int):
    n_idx = t8n * bbn * TILE
    vecs = BLK // L

    mesh = plsc.VectorSubcoreMesh(core_axis_name="c", subcore_axis_name="s")

    @functools.partial(
        pl.kernel,
        mesh=mesh,
        out_type=jax.ShapeDtypeStruct((n_idx * D,), jnp.float32),
        scratch_types=[
            pltpu.VMEM((BLK,), jnp.int32),
            pltpu.VMEM((BLK * D,), jnp.float32),
            pltpu.VMEM((100,), jnp.float32),
        ],
        compiler_params=pltpu.CompilerParams(needs_layout_passes=False),
    )
    def k(idx_hbm, tbl_hbm, out_hbm, idx_v, out_v, tbl_v):
        wid = lax.axis_index("s") * NC + lax.axis_index("c")

        pltpu.sync_copy(tbl_hbm, tbl_v)

        def vec_body(kv, carry):
            vidx = idx_v[pl.ds(kv * L, L)]
            for d in range(D):
                val = plsc.load_gather(tbl_v, [vidx + (d * D)])
                out_v[pl.ds(d * BLK + kv * L, L)] = val + val
            return carry

        def blk_body(t8, carry):
            x_off = (t8 * bbn + wid * GRP) * TILE
            pltpu.sync_copy(idx_hbm.at[pl.ds(x_off, BLK)], idx_v)
            lax.fori_loop(0, vecs, vec_body, 0)
            for d in range(D):
                out_off = ((d * t8n + t8) * bbn + wid * GRP) * TILE
                pltpu.sync_copy(
                    out_v.at[pl.ds(d * BLK, BLK)],
                    out_hbm.at[pl.ds(out_off, BLK)],
                )
            return carry

        lax.fori_loop(0, t8n, blk_body, 0)

    return k


def kernel(x, W):
    b, t = x.shape
    bbn = b // 128
    t8n = t // 8
    xq = x.reshape(bbn, 128, t8n, 8)
    xp = xq.transpose(2, 0, 3, 1).reshape(-1).astype(jnp.int32)
    wt = jnp.transpose(W).reshape(-1).astype(jnp.float32)
    out_flat = _build(t8n, bbn)(xp, wt)
    out5 = out_flat.reshape(D, t8n, bbn, 8, 128)
    return out5.transpose(2, 4, 1, 3, 0).reshape(b, t, D)


# speedup vs baseline: 95.7945x; 3.3701x over previous
import functools

import jax
import jax.numpy as jnp
from jax import lax
from jax.experimental import pallas as pl
from jax.experimental.pallas import tpu as pltpu
from jax.experimental.pallas import tpu_sc as plsc

NC = 2
NS = 16
L = 16
NW = NC * NS

D = 10
TILE = 1024
GRP = 4
BLK = GRP * TILE


@functools.lru_cache(maxsize=None)
def _build(t8n: int, bbn: int):
    n_idx = t8n * bbn * TILE
    vecs = BLK // L

    mesh = plsc.VectorSubcoreMesh(core_axis_name="c", subcore_axis_name="s")

    @functools.partial(
        pl.kernel,
        mesh=mesh,
        out_type=jax.ShapeDtypeStruct((n_idx * D,), jnp.float32),
        scratch_types=[
            pltpu.VMEM((BLK,), jnp.int32),
            pltpu.VMEM((BLK * D,), jnp.float32),
            pltpu.VMEM((100,), jnp.float32),
        ],
        compiler_params=pltpu.CompilerParams(needs_layout_passes=False),
    )
    def k(idx_hbm, tbl_hbm, out_hbm, idx_v, out_v, tbl_v):
        wid = lax.axis_index("s") * NC + lax.axis_index("c")

        pltpu.sync_copy(tbl_hbm, tbl_v)

        def blk_body(t8, carry):
            x_off = (t8 * bbn + wid * GRP) * TILE
            pltpu.sync_copy(idx_hbm.at[pl.ds(x_off, BLK)], idx_v)

            @plsc.parallel_loop(0, vecs, unroll=4)
            def vec_body(kv):
                vidx = idx_v[pl.ds(kv * L, L)]
                for d in range(D):
                    val = plsc.load_gather(tbl_v, [vidx + (d * D)])
                    out_v[pl.ds(d * BLK + kv * L, L)] = val + val

            for d in range(D):
                out_off = ((d * t8n + t8) * bbn + wid * GRP) * TILE
                pltpu.sync_copy(
                    out_v.at[pl.ds(d * BLK, BLK)],
                    out_hbm.at[pl.ds(out_off, BLK)],
                )
            return carry

        lax.fori_loop(0, t8n, blk_body, 0)

    return k


def kernel(x, W):
    b, t = x.shape
    bbn = b // 128
    t8n = t // 8
    xq = x.reshape(bbn, 128, t8n, 8)
    xp = xq.transpose(2, 0, 3, 1).reshape(-1).astype(jnp.int32)
    wt = jnp.transpose(W).reshape(-1).astype(jnp.float32)
    out_flat = _build(t8n, bbn)(xp, wt)
    out5 = out_flat.reshape(D, t8n, bbn, 8, 128)
    return out5.transpose(2, 4, 1, 3, 0).reshape(b, t, D)


# speedup vs baseline: 174.5840x; 1.8225x over previous
import functools

import jax
import jax.numpy as jnp
from jax import lax
from jax.experimental import pallas as pl
from jax.experimental.pallas import tpu as pltpu
from jax.experimental.pallas import tpu_sc as plsc

NC = 2
NS = 16
L = 16
NW = NC * NS

D = 10
TILE = 1024
GRP = 4
BLK = GRP * TILE


@functools.lru_cache(maxsize=None)
def _build(t8n: int, bbn: int):
    n_idx = t8n * bbn * TILE
    vecs = BLK // L

    mesh = plsc.VectorSubcoreMesh(core_axis_name="c", subcore_axis_name="s")

    @functools.partial(
        pl.kernel,
        mesh=mesh,
        out_type=jax.ShapeDtypeStruct((n_idx * D,), jnp.float32),
        scratch_types=[
            pltpu.VMEM((2 * BLK,), jnp.int32),
            pltpu.VMEM((2 * D * BLK,), jnp.float32),
            pltpu.VMEM((100,), jnp.float32),
            pltpu.SemaphoreType.DMA((2,)),
            pltpu.SemaphoreType.DMA((2,)),
        ],
        compiler_params=pltpu.CompilerParams(needs_layout_passes=False),
    )
    def k(idx_hbm, tbl_hbm, out_hbm, idx_v, out_v, tbl_v, in_sems, out_sems):
        wid = lax.axis_index("s") * NC + lax.axis_index("c")

        pltpu.sync_copy(tbl_hbm, tbl_v)

        def in_copy(t8, b):
            x_off = (t8 * bbn + wid * GRP) * TILE
            return pltpu.make_async_copy(
                idx_hbm.at[pl.ds(x_off, BLK)],
                idx_v.at[pl.ds(b * BLK, BLK)],
                in_sems.at[b],
            )

        def out_copy(t8, b, d):
            out_off = ((d * t8n + t8) * bbn + wid * GRP) * TILE
            return pltpu.make_async_copy(
                out_v.at[pl.ds((b * D + d) * BLK, BLK)],
                out_hbm.at[pl.ds(out_off, BLK)],
                out_sems.at[b],
            )

        in_copy(0, 0).start()

        def blk_body(t8, carry):
            b = lax.rem(t8, 2)
            in_copy(t8, b).wait()

            @pl.when(t8 + 1 < t8n)
            def _():
                in_copy(t8 + 1, 1 - b).start()

            @pl.when(t8 >= 2)
            def _():
                for d in range(D):
                    out_copy(t8 - 2, b, d).wait()

            obase = b * D * BLK

            @plsc.parallel_loop(0, vecs, unroll=4)
            def vec_body(kv):
                vidx = idx_v[pl.ds(b * BLK + kv * L, L)]
                for d in range(D):
                    val = plsc.load_gather(tbl_v, [vidx + (d * D)])
                    out_v[pl.ds(obase + d * BLK + kv * L, L)] = val + val

            for d in range(D):
                out_copy(t8, b, d).start()
            return carry

        lax.fori_loop(0, t8n, blk_body, 0)
        for d in range(D):
            out_copy(t8n - 2, (t8n - 2) % 2, d).wait()
            out_copy(t8n - 1, (t8n - 1) % 2, d).wait()

    return k


def kernel(x, W):
    b, t = x.shape
    bbn = b // 128
    t8n = t // 8
    xq = x.reshape(bbn, 128, t8n, 8)
    xp = xq.transpose(2, 0, 3, 1).reshape(-1).astype(jnp.int32)
    wt = jnp.transpose(W).reshape(-1).astype(jnp.float32)
    out_flat = _build(t8n, bbn)(xp, wt)
    out5 = out_flat.reshape(D, t8n, bbn, 8, 128)
    return out5.transpose(2, 4, 1, 3, 0).reshape(b, t, D)


# speedup vs baseline: 176.4300x; 1.0106x over previous
import functools

import jax
import jax.numpy as jnp
from jax import lax
from jax.experimental import pallas as pl
from jax.experimental.pallas import tpu as pltpu
from jax.experimental.pallas import tpu_sc as plsc

NC = 2
NS = 16
L = 16
NW = NC * NS

D = 10
TILE = 1024
GRP = 4
BLK = GRP * TILE


@functools.lru_cache(maxsize=None)
def _build(t8n: int, bbn: int):
    n_idx = t8n * bbn * TILE
    vecs = BLK // L

    mesh = plsc.VectorSubcoreMesh(core_axis_name="c", subcore_axis_name="s")

    @functools.partial(
        pl.kernel,
        mesh=mesh,
        out_type=jax.ShapeDtypeStruct((n_idx * D,), jnp.float32),
        scratch_types=[
            pltpu.VMEM((2 * BLK,), jnp.int32),
            pltpu.VMEM((2 * D * BLK,), jnp.float32),
            pltpu.VMEM((112,), jnp.float32),
            pltpu.SemaphoreType.DMA((2,)),
            pltpu.SemaphoreType.DMA((2,)),
        ],
        compiler_params=pltpu.CompilerParams(needs_layout_passes=False),
    )
    def k(idx_hbm, tbl_hbm, out_hbm, idx_v, out_v, tbl_v, in_sems, out_sems):
        wid = lax.axis_index("s") * NC + lax.axis_index("c")

        pltpu.sync_copy(tbl_hbm, tbl_v.at[pl.ds(0, 100)])
        cols2 = []
        for d in range(D):
            c = tbl_v[pl.ds(d * D, L)]
            cols2.append(c + c)

        gather_dnums = lax.GatherDimensionNumbers(
            offset_dims=(), collapsed_slice_dims=(0,), start_index_map=(0,)
        )

        def dyn_gather(src, idx):
            return lax.gather(
                src,
                idx[:, None],
                gather_dnums,
                slice_sizes=(1,),
                mode=lax.GatherScatterMode.PROMISE_IN_BOUNDS,
            )

        def in_copy(t8, b):
            x_off = (t8 * bbn + wid * GRP) * TILE
            return pltpu.make_async_copy(
                idx_hbm.at[pl.ds(x_off, BLK)],
                idx_v.at[pl.ds(b * BLK, BLK)],
                in_sems.at[b],
            )

        def out_copy(t8, b, d):
            out_off = ((d * t8n + t8) * bbn + wid * GRP) * TILE
            return pltpu.make_async_copy(
                out_v.at[pl.ds((b * D + d) * BLK, BLK)],
                out_hbm.at[pl.ds(out_off, BLK)],
                out_sems.at[b],
            )

        in_copy(0, 0).start()

        def blk_body(t8, carry):
            b = lax.rem(t8, 2)
            in_copy(t8, b).wait()

            @pl.when(t8 + 1 < t8n)
            def _():
                in_copy(t8 + 1, 1 - b).start()

            @pl.when(t8 >= 2)
            def _():
                for d in range(D):
                    out_copy(t8 - 2, b, d).wait()

            obase = b * D * BLK

            @plsc.parallel_loop(0, vecs, unroll=4)
            def vec_body(kv):
                vidx = idx_v[pl.ds(b * BLK + kv * L, L)]
                for d in range(D):
                    out_v[pl.ds(obase + d * BLK + kv * L, L)] = dyn_gather(
                        cols2[d], vidx
                    )

            for d in range(D):
                out_copy(t8, b, d).start()
            return carry

        lax.fori_loop(0, t8n, blk_body, 0)
        for d in range(D):
            out_copy(t8n - 2, (t8n - 2) % 2, d).wait()
            out_copy(t8n - 1, (t8n - 1) % 2, d).wait()

    return k


def kernel(x, W):
    b, t = x.shape
    bbn = b // 128
    t8n = t // 8
    xq = x.reshape(bbn, 128, t8n, 8)
    xp = xq.transpose(2, 0, 3, 1).reshape(-1).astype(jnp.int32)
    wt = jnp.transpose(W).reshape(-1).astype(jnp.float32)
    out_flat = _build(t8n, bbn)(xp, wt)
    out5 = out_flat.reshape(D, t8n, bbn, 8, 128)
    return out5.transpose(2, 4, 1, 3, 0).reshape(b, t, D)
